# baseline (speedup 1.0000x reference)
import jax
import jax.numpy as jnp
from jax.experimental import pallas as pl
from jax.experimental.pallas import tpu as pltpu


def _body(tab_ref, out_ref):
    out_ref[...] = jnp.broadcast_to(tab_ref[...][None, :, :], out_ref.shape)


def kernel(batch_ref, table):
    B = batch_ref.shape[0]
    Q, H = table.shape
    BB = 8
    return pl.pallas_call(
        _body,
        grid=(B // BB,),
        in_specs=[pl.BlockSpec((Q, H), lambda i: (0, 0))],
        out_specs=pl.BlockSpec((BB, Q, H), lambda i: (i, 0, 0)),
        out_shape=jax.ShapeDtypeStruct((B, Q, H), table.dtype),
        compiler_params=pltpu.CompilerParams(
            dimension_semantics=("parallel",),
        ),
    )(table)
